# baseline (speedup 1.0000x reference)
import functools

import jax
import jax.numpy as jnp
from jax import lax
from jax.experimental import pallas as pl
from jax.experimental.pallas import tpu as pltpu
from jax.experimental.pallas import tpu_sc as plsc

_B, _K, _D = 4096, 8192, 32
_BB = 256
_NW = 32
_BPW = _B // _NW


def _sc_gather_body(table_hbm, idx_hbm, out_hbm, idx_v, rows_v, sem):
    wid = lax.axis_index("s") * 2 + lax.axis_index("c")
    base = wid * _BPW
    pltpu.sync_copy(idx_hbm.at[pl.ds(base, _BPW)], idx_v)
    pltpu.async_copy(table_hbm.at[idx_v], rows_v, sem).wait()
    pltpu.sync_copy(rows_v, out_hbm.at[pl.ds(base, _BPW)])


_sc_gather = pl.kernel(
    _sc_gather_body,
    out_type=jax.ShapeDtypeStruct((_B, _D), jnp.float32),
    mesh=plsc.VectorSubcoreMesh(core_axis_name="c", subcore_axis_name="s"),
    scratch_types=[
        pltpu.VMEM((_BPW,), jnp.int32),
        pltpu.VMEM((_BPW, _D), jnp.float32),
        pltpu.SemaphoreType.DMA,
    ],
    compiler_params=pltpu.CompilerParams(use_tc_tiling_on_sc=False),
)


def _tc_body(logits_ref, codes_ref, codebook_ref, soft_ref, lp_ref, ent_ref):
    x = logits_ref[...]
    m = jnp.max(x, axis=1, keepdims=True)
    e = jnp.exp(x - m)
    s = jnp.sum(e, axis=1, keepdims=True)
    t = jnp.sum(e * x, axis=1, keepdims=True)
    logs = jnp.log(s)

    codes = codes_ref[...]
    iota = lax.broadcasted_iota(jnp.int32, (_BB, _K), 1)
    oh = iota == codes
    l_code = jnp.sum(jnp.where(oh, x, 0.0), axis=1, keepdims=True)

    cb = codebook_ref[...]
    dn = (((1,), (0,)), ((), ()))
    v = lax.dot_general(e.astype(jnp.bfloat16), cb, dn,
                        preferred_element_type=jnp.float32)

    soft_ref[...] = v / s
    lp_ref[...] = l_code - m - logs
    ent_ref[...] = m + logs - t / s


@jax.jit
def kernel(logits, codes, codebook):
    hard = _sc_gather(codebook, codes)
    soft, lp, ent = pl.pallas_call(
        _tc_body,
        grid=(_B // _BB,),
        in_specs=[
            pl.BlockSpec((_BB, _K), lambda i: (i, 0)),
            pl.BlockSpec((_BB, 1), lambda i: (i, 0)),
            pl.BlockSpec((_K, _D), lambda i: (0, 0)),
        ],
        out_specs=[
            pl.BlockSpec((_BB, _D), lambda i: (i, 0)),
            pl.BlockSpec((_BB, 1), lambda i: (i, 0)),
            pl.BlockSpec((_BB, 1), lambda i: (i, 0)),
        ],
        out_shape=[
            jax.ShapeDtypeStruct((_B, _D), jnp.float32),
            jax.ShapeDtypeStruct((_B, 1), jnp.float32),
            jax.ShapeDtypeStruct((_B, 1), jnp.float32),
        ],
    )(logits, codes.reshape(_B, 1), codebook.astype(jnp.bfloat16))
    return jnp.concatenate([hard, soft, lp, ent], axis=-1)
